# baseline (speedup 1.0000x reference)
import jax
import jax.numpy as jnp
from jax.experimental import pallas as pl
from jax.experimental.pallas import tpu as pltpu

_C = 14
_GAMMA = 2.0
_MULT_UNLABELED = 3.0
_EPS = 1e-06
_LANE = 128
_TM = 1728


def _body(annot_ref, probs_a, probs_b, target_a, target_b, out_ref,
          ent_acc, ce_acc, fg_acc):
    b = pl.program_id(0)
    j = pl.program_id(1)
    nj = pl.num_programs(1)

    @pl.when(j == 0)
    def _init():
        ent_acc[...] = jnp.zeros_like(ent_acc)
        ce_acc[...] = jnp.zeros_like(ce_acc)
        fg_acc[0] = 0

    un = []
    for c in range(1, _C):
        pres = annot_ref[b, 0] == c
        for k in range(1, annot_ref.shape[1]):
            pres = pres | (annot_ref[b, k] == c)
        un.append(jnp.where(pres, 0.0, 1.0))

    ent_t = jnp.zeros((8, _LANE), jnp.float32)
    ce_t = jnp.zeros((8, _LANE), jnp.float32)
    fg_m = None
    for probs_ref, target_ref in ((probs_a, target_a), (probs_b, target_b)):
        for g in range(_TM // 8):
            sl = slice(g * 8, g * 8 + 8)
            t_v = target_ref[0, sl, :]
            p0 = probs_ref[0, 0, sl, :]
            ent_g = p0 * jnp.log(jnp.clip(p0, _EPS, 1.0 - _EPS))
            qfg = p0
            sun = p0
            for c in range(1, _C):
                p_c = probs_ref[0, c, sl, :]
                ent_g = ent_g + p_c * jnp.log(jnp.clip(p_c, _EPS, 1.0 - _EPS))
                qfg = jnp.where(t_v == c, p_c, qfg)
                sun = sun + p_c * un[c - 1]
            q = jnp.where(t_v == 0, sun, qfg)
            omq = 1.0 - q
            ce_t = ce_t - (omq * omq) * jnp.log(jnp.clip(q, _EPS, 1.0 - _EPS))
            ent_t = ent_t + ent_g
        tm = jnp.max(target_ref[0])
        fg_m = tm if fg_m is None else jnp.maximum(fg_m, tm)

    ent_acc[...] += ent_t
    ce_acc[...] += ce_t
    fg_acc[0] = jnp.maximum(fg_acc[0], fg_m)

    @pl.when(j == nj - 1)
    def _fini():
        out_ref[b, 0] = jnp.sum(ent_acc[...])
        out_ref[b, 1] = jnp.sum(ce_acc[...])
        out_ref[b, 2] = fg_acc[0].astype(jnp.float32)


def kernel(probs, target, annotated_fg_categories):
    B, C = probs.shape[0], probs.shape[1]
    n_vox = probs.shape[2] * probs.shape[3] * probs.shape[4]
    M = n_vox // _LANE
    nj = M // (2 * _TM)

    p4 = probs.reshape(B, C, M, _LANE)
    t3 = target.reshape(B, M, _LANE)

    out = pl.pallas_call(
        _body,
        grid=(B, nj),
        in_specs=[
            pl.BlockSpec(memory_space=pltpu.SMEM),
            pl.BlockSpec((1, C, _TM, _LANE), lambda b, j: (b, 0, 2 * j, 0)),
            pl.BlockSpec((1, C, _TM, _LANE), lambda b, j: (b, 0, 2 * j + 1, 0)),
            pl.BlockSpec((1, _TM, _LANE), lambda b, j: (b, 2 * j, 0)),
            pl.BlockSpec((1, _TM, _LANE), lambda b, j: (b, 2 * j + 1, 0)),
        ],
        out_specs=pl.BlockSpec(memory_space=pltpu.SMEM),
        out_shape=jax.ShapeDtypeStruct((B, 3), jnp.float32),
        scratch_shapes=[
            pltpu.VMEM((8, _LANE), jnp.float32),
            pltpu.VMEM((8, _LANE), jnp.float32),
            pltpu.SMEM((1,), jnp.int32),
        ],
    )(annotated_fg_categories, p4, p4, t3, t3)

    ent_sum = out[:, 0]
    ce_sum = out[:, 1]
    tmax = out[:, 2]
    nf = jnp.float32(n_vox)
    mult = jnp.where(tmax > 0.0, 1.0, _MULT_UNLABELED)
    reg = -jnp.sum(mult * (ent_sum / nf)) / B
    ce = jnp.mean(ce_sum / nf)
    return ce, reg


# speedup vs baseline: 1.0095x; 1.0095x over previous
import jax
import jax.numpy as jnp
from jax.experimental import pallas as pl
from jax.experimental.pallas import tpu as pltpu

_C = 14
_GAMMA = 2.0
_MULT_UNLABELED = 3.0
_EPS = 1e-06
_LANE = 128
_TM = 864


def _body(annot_ref, probs_a, probs_b, target_a, target_b, out_ref,
          ent_acc, ce_acc, fg_acc):
    b = pl.program_id(0)
    j = pl.program_id(1)
    nj = pl.num_programs(1)

    @pl.when(j == 0)
    def _init():
        ent_acc[...] = jnp.zeros_like(ent_acc)
        ce_acc[...] = jnp.zeros_like(ce_acc)
        fg_acc[0] = 0

    un = []
    for c in range(1, _C):
        pres = annot_ref[b, 0] == c
        for k in range(1, annot_ref.shape[1]):
            pres = pres | (annot_ref[b, k] == c)
        un.append(jnp.where(pres, 0.0, 1.0))

    ent_t = jnp.zeros((8, _LANE), jnp.float32)
    ce_t = jnp.zeros((8, _LANE), jnp.float32)
    fg_m = None
    for probs_ref, target_ref in ((probs_a, target_a), (probs_b, target_b)):
        for g in range(_TM // 8):
            sl = slice(g * 8, g * 8 + 8)
            t_v = target_ref[0, sl, :]
            p0 = probs_ref[0, 0, sl, :]
            ent_g = p0 * jnp.log(jnp.clip(p0, _EPS, 1.0 - _EPS))
            qfg = p0
            sun = p0
            for c in range(1, _C):
                p_c = probs_ref[0, c, sl, :]
                ent_g = ent_g + p_c * jnp.log(jnp.clip(p_c, _EPS, 1.0 - _EPS))
                qfg = jnp.where(t_v == c, p_c, qfg)
                sun = sun + p_c * un[c - 1]
            q = jnp.where(t_v == 0, sun, qfg)
            omq = 1.0 - q
            ce_t = ce_t - (omq * omq) * jnp.log(jnp.clip(q, _EPS, 1.0 - _EPS))
            ent_t = ent_t + ent_g
        tm = jnp.max(target_ref[0])
        fg_m = tm if fg_m is None else jnp.maximum(fg_m, tm)

    ent_acc[...] += ent_t
    ce_acc[...] += ce_t
    fg_acc[0] = jnp.maximum(fg_acc[0], fg_m)

    @pl.when(j == nj - 1)
    def _fini():
        out_ref[b, 0] = jnp.sum(ent_acc[...])
        out_ref[b, 1] = jnp.sum(ce_acc[...])
        out_ref[b, 2] = fg_acc[0].astype(jnp.float32)


def kernel(probs, target, annotated_fg_categories):
    B, C = probs.shape[0], probs.shape[1]
    n_vox = probs.shape[2] * probs.shape[3] * probs.shape[4]
    M = n_vox // _LANE
    nj = M // (2 * _TM)

    p4 = probs.reshape(B, C, M, _LANE)
    t3 = target.reshape(B, M, _LANE)

    out = pl.pallas_call(
        _body,
        grid=(B, nj),
        in_specs=[
            pl.BlockSpec(memory_space=pltpu.SMEM),
            pl.BlockSpec((1, C, _TM, _LANE), lambda b, j: (b, 0, 2 * j, 0)),
            pl.BlockSpec((1, C, _TM, _LANE), lambda b, j: (b, 0, 2 * j + 1, 0)),
            pl.BlockSpec((1, _TM, _LANE), lambda b, j: (b, 2 * j, 0)),
            pl.BlockSpec((1, _TM, _LANE), lambda b, j: (b, 2 * j + 1, 0)),
        ],
        out_specs=pl.BlockSpec(memory_space=pltpu.SMEM),
        out_shape=jax.ShapeDtypeStruct((B, 3), jnp.float32),
        scratch_shapes=[
            pltpu.VMEM((8, _LANE), jnp.float32),
            pltpu.VMEM((8, _LANE), jnp.float32),
            pltpu.SMEM((1,), jnp.int32),
        ],
    )(annotated_fg_categories, p4, p4, t3, t3)

    ent_sum = out[:, 0]
    ce_sum = out[:, 1]
    tmax = out[:, 2]
    nf = jnp.float32(n_vox)
    mult = jnp.where(tmax > 0.0, 1.0, _MULT_UNLABELED)
    reg = -jnp.sum(mult * (ent_sum / nf)) / B
    ce = jnp.mean(ce_sum / nf)
    return ce, reg
